# baseline (speedup 1.0000x reference)
import jax
import jax.numpy as jnp
import numpy as _np
from jax import lax
from jax.experimental import pallas as pl
from jax.experimental.pallas import tpu as pltpu
from jax.experimental.pallas import tpu_sc as plsc

N = 50000
E = 800000
H = 64
HH = 32

NP = 50048
EP = 819200
NTILES = 16
ROWS_PER_TILE = NP // NTILES
EROWS = EP // 128
EROWS_PER_TILE = EROWS // NTILES
C = 2
NC = EROWS_PER_TILE // C
NCHUNKS = EROWS // C

BN = 2944
GRID = NP // BN

_QH = _np.concatenate([_np.arange(0, 32, 2), _np.arange(1, 32, 2)])
_QF = _np.concatenate([_QH, 32 + _QH])

_SC_MESH = plsc.VectorSubcoreMesh(core_axis_name="c", subcore_axis_name="s")
_SC_PARAMS = pltpu.CompilerParams(use_tc_tiling_on_sc=False,
                                  needs_layout_passes=False)


def _one_pass(c, s, rbase, deg, t0b, t1b, t0f, t1f, idxc, out0, out1,
              idx, rows_b, rows, acc, sem_i0, sem_i1, sem_g0, sem_g1, sem_s):
    @pl.when(c == 0)
    def _():
        pltpu.sync_copy(t0f.at[pl.ds(rbase, ROWS_PER_TILE)],
                        acc.at[pl.ds(rbase, ROWS_PER_TILE)])

    @pl.when(c == 1)
    def _():
        pltpu.sync_copy(t1f.at[pl.ds(rbase, ROWS_PER_TILE)],
                        acc.at[pl.ds(rbase, ROWS_PER_TILE)])

    @pl.when(deg)
    def _():
        for m in range(2):
            for j in range(C):
                pltpu.sync_copy(t0f.at[pl.ds(0, 128)], rows.at[m, j])

    plsc.subcore_barrier()

    not_deg = jnp.logical_not(deg)
    cbase = s * NC

    def idx_load(k, sem):
        m = lax.rem(k, 4) if not isinstance(k, int) else (k % 4)
        pltpu.async_copy(idxc.at[cbase + k], idx.at[m], sem)

    def drain_idx(sem):
        pltpu.make_async_copy(idxc.at[0], idx.at[0], sem).wait()

    def fire_gather(k, sem):
        m4 = lax.rem(k, 4) if not isinstance(k, int) else (k % 4)
        m3 = lax.rem(k, 3) if not isinstance(k, int) else (k % 3)

        @pl.when(jnp.logical_and(not_deg, c == 0))
        def _():
            for j in range(C):
                pltpu.async_copy(t0b.at[idx.at[m4, 0, j]], rows_b.at[m3, j],
                                 sem)

        @pl.when(jnp.logical_and(not_deg, c == 1))
        def _():
            for j in range(C):
                pltpu.async_copy(t1b.at[idx.at[m4, 0, j]], rows_b.at[m3, j],
                                 sem)

    def drain_gather(sem):
        @pl.when(not_deg)
        def _():
            for j in range(C):
                pltpu.make_async_copy(t0b.at[pl.ds(0, 128)], rows_b.at[0, j],
                                      sem).wait()

    def convert(k):
        m3 = lax.rem(k, 3) if not isinstance(k, int) else (k % 3)
        m2 = lax.rem(k, 2) if not isinstance(k, int) else (k % 2)

        @pl.when(not_deg)
        def _():
            def cv(g, carry):
                for u in range(4):
                    r = g * 4 + u
                    for j in range(C):
                        v = rows_b[m3, j, r]
                        a, b = plsc.unpack(
                            v, format=plsc.PackFormat.INTERLEAVED,
                            preferred_element_type=jnp.float32)
                        rows[m2, j, r, 0:16] = a
                        rows[m2, j, r, 16:32] = b
                return carry

            lax.fori_loop(0, 32, cv, 0)

    def fire_scatter(k):
        m4 = lax.rem(k, 4) if not isinstance(k, int) else (k % 4)
        m2 = lax.rem(k, 2) if not isinstance(k, int) else (k % 2)
        for j in range(C):
            pltpu.async_copy(rows.at[m2, j], acc.at[idx.at[m4, 1, j]],
                             sem_s, add=True)

    def drain_scatter():
        for j in range(C):
            pltpu.make_async_copy(t0f.at[pl.ds(0, 128)], rows.at[0, j],
                                  sem_s).wait()

    idx_load(0, sem_i0)
    drain_idx(sem_i0)
    idx_load(1, sem_i1)
    drain_idx(sem_i1)
    fire_gather(0, sem_g0)
    idx_load(2, sem_i0)
    fire_gather(1, sem_g1)

    def halfstep(k, sg_k, si_k, si_k1):
        drain_gather(sg_k)
        convert(k)

        @pl.when(k >= 1)
        def _():
            drain_scatter()

        fire_scatter(k)

        @pl.when(k + 2 < NC)
        def _():
            drain_idx(si_k)

        @pl.when(k + 3 < NC)
        def _():
            idx_load(k + 3, si_k1)

        @pl.when(k + 2 < NC)
        def _():
            fire_gather(k + 2, sg_k)

    def step(i, carry):
        k = i * 2
        halfstep(k, sem_g0, sem_i0, sem_i1)
        halfstep(k + 1, sem_g1, sem_i1, sem_i0)
        return carry

    lax.fori_loop(0, NC // 2, step, 0)
    drain_scatter()
    plsc.subcore_barrier()

    @pl.when(c == 0)
    def _():
        pltpu.sync_copy(acc.at[pl.ds(rbase, ROWS_PER_TILE)],
                        out0.at[pl.ds(rbase, ROWS_PER_TILE)])

    @pl.when(c == 1)
    def _():
        pltpu.sync_copy(acc.at[pl.ds(rbase, ROWS_PER_TILE)],
                        out1.at[pl.ds(rbase, ROWS_PER_TILE)])


def _spmm_body(ta0b, ta1b, ta0f, ta1f, tb0b, tb1b, tb0f, tb1f, idxc, flags,
               oa0, oa1, ob0, ob1,
               idx, rows_b, rows, acc,
               sem_i0, sem_i1, sem_g0, sem_g1, sem_s):
    c = lax.axis_index("c")
    s = lax.axis_index("s")
    rbase = s * ROWS_PER_TILE

    pltpu.sync_copy(flags, idx.at[0, 0, 0, pl.ds(0, 16)])
    fv = idx[0, 0, 0, 0:16]
    lane = lax.iota(jnp.int32, 16)
    do_b = jnp.sum(jnp.where(lane == 0, fv, 0)) > 0
    deg = jnp.sum(jnp.where(lane == 1, fv, 0)) > 0

    _one_pass(c, s, rbase, deg, ta0b, ta1b, ta0f, ta1f, idxc, oa0, oa1,
              idx, rows_b, rows, acc, sem_i0, sem_i1, sem_g0, sem_g1, sem_s)

    @pl.when(do_b)
    def _():
        _one_pass(c, s, rbase, deg, tb0b, tb1b, tb0f, tb1f, idxc, ob0, ob1,
                  idx, rows_b, rows, acc, sem_i0, sem_i1, sem_g0, sem_g1,
                  sem_s)


_sc_spmm = pl.kernel(
    _spmm_body,
    out_type=tuple(jax.ShapeDtypeStruct((NP, HH), jnp.float32)
                   for _ in range(4)),
    mesh=_SC_MESH,
    scratch_types=[
        pltpu.VMEM((4, 2, C, 128), jnp.int32),
        pltpu.VMEM((3, C, 128, HH), jnp.bfloat16),
        pltpu.VMEM((2, C, 128, HH), jnp.float32),
        pltpu.VMEM_SHARED((NP, HH), jnp.float32),
        pltpu.SemaphoreType.DMA,
        pltpu.SemaphoreType.DMA,
        pltpu.SemaphoreType.DMA,
        pltpu.SemaphoreType.DMA,
        pltpu.SemaphoreType.DMA,
    ],
    compiler_params=_SC_PARAMS,
)


def _row_spec(w):
    return pl.BlockSpec((BN, w), lambda i: (i, 0))


def _full_spec(shape):
    return pl.BlockSpec(shape, lambda i: tuple(0 for _ in shape))


def _relu(v):
    return jnp.maximum(v, 0.0)


def _dot(a, b):
    return jnp.dot(a, b, preferred_element_type=jnp.float32)


def _emit_tables(hf, hb, outs):
    f0, f1, b0, b1 = outs
    f0[...] = hf[:, :HH]
    f1[...] = hf[:, HH:]
    b0[...] = hb[:, :HH].astype(jnp.bfloat16)
    b1[...] = hb[:, HH:].astype(jnp.bfloat16)


def _enc_body(x, pin, d0, W_se, b_se, W_pe1, b_pe1, W_pe2, b_pe2,
              W_c1, W_c1q, W_pc1, W_pc1q,
              dinv_o, tsf0, tsf1, tsb0, tsb1, tpf0, tpf1, tpb0, tpb1):
    dinv = lax.rsqrt(d0[:, :1])
    s0 = _relu(x[:, 0:1] * W_se[0:1, :] + x[:, 1:2] * W_se[1:2, :] + b_se[...])
    p = _relu(pin[:, 0:1] * W_pe1[0:1, :] + pin[:, 1:2] * W_pe1[1:2, :]
              + b_pe1[...])
    p0 = _relu(_dot(p, W_pe2[...]) + b_pe2[...])
    dinv_o[...] = dinv
    _emit_tables(dinv * _dot(s0, W_c1q[...]), dinv * _dot(s0, W_c1[...]),
                 (tsf0, tsf1, tsb0, tsb1))
    _emit_tables(dinv * _dot(p0, W_pc1q[...]), dinv * _dot(p0, W_pc1[...]),
                 (tpf0, tpf1, tpb0, tpb1))


def _tc_enc(*args):
    return pl.pallas_call(
        _enc_body,
        grid=(GRID,),
        in_specs=[_row_spec(2), _row_spec(2), _row_spec(HH),
                  _full_spec((2, H)), _full_spec((1, H)),
                  _full_spec((2, H)), _full_spec((1, H)),
                  _full_spec((H, H)), _full_spec((1, H)),
                  _full_spec((H, H)), _full_spec((H, H)),
                  _full_spec((H, H)), _full_spec((H, H))],
        out_specs=[_row_spec(1)] + [_row_spec(HH)] * 8,
        out_shape=[jax.ShapeDtypeStruct((NP, 1), jnp.float32)]
        + [jax.ShapeDtypeStruct((NP, HH), jnp.float32)] * 2
        + [jax.ShapeDtypeStruct((NP, HH), jnp.bfloat16)] * 2
        + [jax.ShapeDtypeStruct((NP, HH), jnp.float32)] * 2
        + [jax.ShapeDtypeStruct((NP, HH), jnp.bfloat16)] * 2,
    )(*args)


def _round_body(Ss0, Ss1, Sp0, Sp1, dinv, b_s, b_p,
                W_sg, W_sq, W_pg, W_pq,
                tsf0, tsf1, tsb0, tsb1, tpf0, tpf1, tpb0, tpb1):
    dv = dinv[...]
    s = _relu(dv * jnp.concatenate([Ss0[...], Ss1[...]], axis=1) + b_s[...])
    p = _relu(dv * jnp.concatenate([Sp0[...], Sp1[...]], axis=1) + b_p[...])
    _emit_tables(dv * _dot(s, W_sq[...]), dv * _dot(s, W_sg[...]),
                 (tsf0, tsf1, tsb0, tsb1))
    _emit_tables(dv * _dot(p, W_pq[...]), dv * _dot(p, W_pg[...]),
                 (tpf0, tpf1, tpb0, tpb1))


def _tc_round(*args):
    return pl.pallas_call(
        _round_body,
        grid=(GRID,),
        in_specs=[_row_spec(HH)] * 4 + [_row_spec(1),
                  _full_spec((1, H)), _full_spec((1, H))]
        + [_full_spec((H, H))] * 4,
        out_specs=[_row_spec(HH)] * 8,
        out_shape=([jax.ShapeDtypeStruct((NP, HH), jnp.float32)] * 2
                   + [jax.ShapeDtypeStruct((NP, HH), jnp.bfloat16)] * 2) * 2,
    )(*args)


def _mix_body(Ss0, Ss1, Sp0, Sp1, dinv, b_s, b_p,
              W_tg, W_tq, W_bg, W_bq, tcf0, tcf1, tcb0, tcb1):
    dv = dinv[...]
    s = _relu(dv * jnp.concatenate([Ss0[...], Ss1[...]], axis=1) + b_s[...])
    p = _relu(dv * jnp.concatenate([Sp0[...], Sp1[...]], axis=1) + b_p[...])
    hf = dv * (_dot(s, W_tq[...]) + _dot(p, W_bq[...]))
    hb = dv * (_dot(s, W_tg[...]) + _dot(p, W_bg[...]))
    _emit_tables(hf, hb, (tcf0, tcf1, tcb0, tcb1))


def _tc_mix(*args):
    return pl.pallas_call(
        _mix_body,
        grid=(GRID,),
        in_specs=[_row_spec(HH)] * 4 + [_row_spec(1),
                  _full_spec((1, H)), _full_spec((1, H))]
        + [_full_spec((H, H))] * 4,
        out_specs=[_row_spec(HH)] * 4,
        out_shape=[jax.ShapeDtypeStruct((NP, HH), jnp.float32)] * 2
        + [jax.ShapeDtypeStruct((NP, HH), jnp.bfloat16)] * 2,
    )(*args)


def _fin_body(Sc0, Sc1, dinv, b_cc, W_d1, b_d1, W_d2, b_d2, W_d3, b_d3, out):
    dv = dinv[...]
    c = _relu(dv * jnp.concatenate([Sc0[...], Sc1[...]], axis=1) + b_cc[...])
    d = _relu(_dot(c, W_d1[...]) + b_d1[...])
    d = _relu(_dot(d, W_d2[...]) + b_d2[...])
    d3 = jnp.sum(d * W_d3[...], axis=1, keepdims=True) + b_d3[...]
    out[...] = 1.0 / (1.0 + jnp.exp(-d3))


def _tc_fin(*args):
    return pl.pallas_call(
        _fin_body,
        grid=(GRID,),
        in_specs=[_row_spec(HH)] * 2 + [_row_spec(1),
                  _full_spec((1, H)),
                  _full_spec((H, H)), _full_spec((1, H)),
                  _full_spec((H, HH)), _full_spec((1, HH)),
                  _full_spec((1, HH)), _full_spec((1, 1))],
        out_specs=[_row_spec(1)],
        out_shape=[jax.ShapeDtypeStruct((NP, 1), jnp.float32)],
    )(*args)[0]


@jax.jit
def _run(x, true_alpha_t, true_torque_t, edge_index,
         W_se, b_se, W_pe1, b_pe1, W_pe2, b_pe2,
         W_c1, b_c1, W_c2, b_c2, W_pc1, b_pc1, W_pc2, b_pc2,
         W_cc, b_cc, W_d1, b_d1, W_d2, b_d2, W_d3, b_d3):
    f32 = jnp.float32
    i32 = jnp.int32
    qf = jnp.asarray(_QF)
    xp = jnp.zeros((NP, 2), f32).at[:N].set(x)
    pin = jnp.zeros((NP, 2), f32).at[:N, 0].set(true_alpha_t[:, 0])
    pin = pin.at[:N, 1].set(true_torque_t[:, 0])
    pad = jnp.full((EP - E,), N, i32)
    srcr = jnp.concatenate([edge_index[0], pad]).reshape(NCHUNKS, C, 128)
    dstr = jnp.concatenate([edge_index[1], pad]).reshape(NCHUNKS, C, 128)
    idxc = jnp.stack([srcr, dstr], axis=1)
    ones_b = jnp.ones((NP, HH), jnp.bfloat16)
    ones_f = jnp.ones((NP, HH), f32)

    def row(b):
        return b.reshape(1, -1)

    f_deg = jnp.zeros((16,), i32).at[1].set(1)
    f_dual = jnp.zeros((16,), i32).at[0].set(1)
    f_one = jnp.zeros((16,), i32)

    W_c1q, W_pc1q = W_c1[:, qf], W_pc1[:, qf]
    W_c2g, W_c2q = W_c2[qf, :], W_c2[qf][:, qf]
    W_pc2g, W_pc2q = W_pc2[qf, :], W_pc2[qf][:, qf]
    Wcc_t, Wcc_b = W_cc[:H], W_cc[H:]
    Wcc_tg, Wcc_tq = Wcc_t[qf, :], Wcc_t[qf][:, qf]
    Wcc_bg, Wcc_bq = Wcc_b[qf, :], Wcc_b[qf][:, qf]
    b_c1q, b_pc1q = b_c1[qf], b_pc1[qf]
    b_c2q, b_pc2q = b_c2[qf], b_pc2[qf]
    b_ccq = b_cc[qf]
    W_d1g = W_d1[qf, :]

    d0, _, _, _ = _sc_spmm(ones_b, ones_b, ones_f, ones_f,
                           ones_b, ones_b, ones_f, ones_f, idxc, f_deg)

    (dinv, tsf0, tsf1, tsb0, tsb1, tpf0, tpf1, tpb0, tpb1) = _tc_enc(
        xp, pin, d0, W_se, row(b_se), W_pe1, row(b_pe1), W_pe2, row(b_pe2),
        W_c1, W_c1q, W_pc1, W_pc1q)

    Ss0, Ss1, Sp0, Sp1 = _sc_spmm(tsb0, tsb1, tsf0, tsf1,
                                  tpb0, tpb1, tpf0, tpf1, idxc, f_dual)
    (tsf0, tsf1, tsb0, tsb1, tpf0, tpf1, tpb0, tpb1) = _tc_round(
        Ss0, Ss1, Sp0, Sp1, dinv, row(b_c1q), row(b_pc1q),
        W_c2g, W_c2q, W_pc2g, W_pc2q)

    Ss0, Ss1, Sp0, Sp1 = _sc_spmm(tsb0, tsb1, tsf0, tsf1,
                                  tpb0, tpb1, tpf0, tpf1, idxc, f_dual)
    tcf0, tcf1, tcb0, tcb1 = _tc_mix(
        Ss0, Ss1, Sp0, Sp1, dinv, row(b_c2q), row(b_pc2q),
        Wcc_tg, Wcc_tq, Wcc_bg, Wcc_bq)

    Sc0, Sc1, _, _ = _sc_spmm(tcb0, tcb1, tcf0, tcf1,
                              tcb0, tcb1, tcf0, tcf1, idxc, f_one)
    out = _tc_fin(Sc0, Sc1, dinv, row(b_ccq), W_d1g, row(b_d1),
                  W_d2, row(b_d2), W_d3.reshape(1, HH), b_d3.reshape(1, 1))
    return out[:N]


def kernel(x, true_alpha_t, true_torque_t, edge_index,
           W_se, b_se, W_pe1, b_pe1, W_pe2, b_pe2,
           W_c1, b_c1, W_c2, b_c2, W_pc1, b_pc1, W_pc2, b_pc2,
           W_cc, b_cc, W_d1, b_d1, W_d2, b_d2, W_d3, b_d3):
    return _run(x, true_alpha_t, true_torque_t, edge_index,
                W_se, b_se, W_pe1, b_pe1, W_pe2, b_pe2,
                W_c1, b_c1, W_c2, b_c2, W_pc1, b_pc1, W_pc2, b_pc2,
                W_cc, b_cc, W_d1, b_d1, W_d2, b_d2, W_d3, b_d3)


# speedup vs baseline: 1.0245x; 1.0245x over previous
import jax
import jax.numpy as jnp
import numpy as _np
from jax import lax
from jax.experimental import pallas as pl
from jax.experimental.pallas import tpu as pltpu
from jax.experimental.pallas import tpu_sc as plsc

N = 50000
E = 800000
H = 64
HH = 32

NP = 50048
EP = 819200
NTILES = 16
ROWS_PER_TILE = NP // NTILES
EROWS = EP // 128
EROWS_PER_TILE = EROWS // NTILES
C = 2
NC = EROWS_PER_TILE // C
NCHUNKS = EROWS // C

BN = 2944
GRID = NP // BN

_QH = _np.concatenate([_np.arange(0, 32, 2), _np.arange(1, 32, 2)])
_QF = _np.concatenate([_QH, 32 + _QH])

_SC_MESH = plsc.VectorSubcoreMesh(core_axis_name="c", subcore_axis_name="s")
_SC_PARAMS = pltpu.CompilerParams(use_tc_tiling_on_sc=False,
                                  needs_layout_passes=False)


def _one_pass(c, s, rbase, deg, t0b, t1b, t0f, t1f, idxc, out0, out1,
              idx, rows_b, rows, acc, sem_i0, sem_i1, sem_g0, sem_g1, sem_s):
    @pl.when(c == 0)
    def _():
        pltpu.sync_copy(t0f.at[pl.ds(rbase, ROWS_PER_TILE)],
                        acc.at[pl.ds(rbase, ROWS_PER_TILE)])

    @pl.when(c == 1)
    def _():
        pltpu.sync_copy(t1f.at[pl.ds(rbase, ROWS_PER_TILE)],
                        acc.at[pl.ds(rbase, ROWS_PER_TILE)])

    @pl.when(deg)
    def _():
        for m in range(2):
            for j in range(C):
                pltpu.sync_copy(t0f.at[pl.ds(0, 128)], rows.at[m, j])

    plsc.subcore_barrier()

    not_deg = jnp.logical_not(deg)
    cbase = s * NC

    def idx_load(k, sem):
        m = lax.rem(k, 4) if not isinstance(k, int) else (k % 4)
        pltpu.async_copy(idxc.at[cbase + k], idx.at[m], sem)

    def drain_idx(sem):
        pltpu.make_async_copy(idxc.at[0], idx.at[0], sem).wait()

    def fire_gather(k, sem):
        m4 = lax.rem(k, 4) if not isinstance(k, int) else (k % 4)
        m3 = lax.rem(k, 3) if not isinstance(k, int) else (k % 3)

        @pl.when(jnp.logical_and(not_deg, c == 0))
        def _():
            for j in range(C):
                pltpu.async_copy(t0b.at[idx.at[m4, 0, j]], rows_b.at[m3, j],
                                 sem)

        @pl.when(jnp.logical_and(not_deg, c == 1))
        def _():
            for j in range(C):
                pltpu.async_copy(t1b.at[idx.at[m4, 0, j]], rows_b.at[m3, j],
                                 sem)

    def drain_gather(sem):
        @pl.when(not_deg)
        def _():
            for j in range(C):
                pltpu.make_async_copy(t0b.at[pl.ds(0, 128)], rows_b.at[0, j],
                                      sem).wait()

    def convert(k):
        m3 = lax.rem(k, 3) if not isinstance(k, int) else (k % 3)
        m2 = lax.rem(k, 2) if not isinstance(k, int) else (k % 2)

        @pl.when(not_deg)
        def _():
            def cv(r, carry):
                for j in range(C):
                    v = rows_b[m3, j, r]
                    a, b = plsc.unpack(v, format=plsc.PackFormat.INTERLEAVED,
                                       preferred_element_type=jnp.float32)
                    rows[m2, j, r, 0:16] = a
                    rows[m2, j, r, 16:32] = b
                return carry

            lax.fori_loop(0, 128, cv, 0)

    def fire_scatter(k):
        m4 = lax.rem(k, 4) if not isinstance(k, int) else (k % 4)
        m2 = lax.rem(k, 2) if not isinstance(k, int) else (k % 2)
        for j in range(C):
            pltpu.async_copy(rows.at[m2, j], acc.at[idx.at[m4, 1, j]],
                             sem_s, add=True)

    def drain_scatter():
        for j in range(C):
            pltpu.make_async_copy(t0f.at[pl.ds(0, 128)], rows.at[0, j],
                                  sem_s).wait()

    idx_load(0, sem_i0)
    drain_idx(sem_i0)
    idx_load(1, sem_i1)
    drain_idx(sem_i1)
    fire_gather(0, sem_g0)
    idx_load(2, sem_i0)
    fire_gather(1, sem_g1)

    def halfstep(k, sg_k, si_k, si_k1):
        drain_gather(sg_k)
        convert(k)

        @pl.when(k >= 1)
        def _():
            drain_scatter()

        fire_scatter(k)

        @pl.when(k + 2 < NC)
        def _():
            drain_idx(si_k)

        @pl.when(k + 3 < NC)
        def _():
            idx_load(k + 3, si_k1)

        @pl.when(k + 2 < NC)
        def _():
            fire_gather(k + 2, sg_k)

    def step(i, carry):
        k = i * 2
        halfstep(k, sem_g0, sem_i0, sem_i1)
        halfstep(k + 1, sem_g1, sem_i1, sem_i0)
        return carry

    lax.fori_loop(0, NC // 2, step, 0)
    drain_scatter()
    plsc.subcore_barrier()

    @pl.when(c == 0)
    def _():
        pltpu.sync_copy(acc.at[pl.ds(rbase, ROWS_PER_TILE)],
                        out0.at[pl.ds(rbase, ROWS_PER_TILE)])

    @pl.when(c == 1)
    def _():
        pltpu.sync_copy(acc.at[pl.ds(rbase, ROWS_PER_TILE)],
                        out1.at[pl.ds(rbase, ROWS_PER_TILE)])


def _spmm_body(ta0b, ta1b, ta0f, ta1f, tb0b, tb1b, tb0f, tb1f, idxc, flags,
               oa0, oa1, ob0, ob1,
               idx, rows_b, rows, acc,
               sem_i0, sem_i1, sem_g0, sem_g1, sem_s):
    c = lax.axis_index("c")
    s = lax.axis_index("s")
    rbase = s * ROWS_PER_TILE

    pltpu.sync_copy(flags, idx.at[0, 0, 0, pl.ds(0, 16)])
    fv = idx[0, 0, 0, 0:16]
    lane = lax.iota(jnp.int32, 16)
    do_b = jnp.sum(jnp.where(lane == 0, fv, 0)) > 0
    deg = jnp.sum(jnp.where(lane == 1, fv, 0)) > 0

    _one_pass(c, s, rbase, deg, ta0b, ta1b, ta0f, ta1f, idxc, oa0, oa1,
              idx, rows_b, rows, acc, sem_i0, sem_i1, sem_g0, sem_g1, sem_s)

    @pl.when(do_b)
    def _():
        _one_pass(c, s, rbase, deg, tb0b, tb1b, tb0f, tb1f, idxc, ob0, ob1,
                  idx, rows_b, rows, acc, sem_i0, sem_i1, sem_g0, sem_g1,
                  sem_s)


_sc_spmm = pl.kernel(
    _spmm_body,
    out_type=tuple(jax.ShapeDtypeStruct((NP, HH), jnp.float32)
                   for _ in range(4)),
    mesh=_SC_MESH,
    scratch_types=[
        pltpu.VMEM((4, 2, C, 128), jnp.int32),
        pltpu.VMEM((3, C, 128, HH), jnp.bfloat16),
        pltpu.VMEM((2, C, 128, HH), jnp.float32),
        pltpu.VMEM_SHARED((NP, HH), jnp.float32),
        pltpu.SemaphoreType.DMA,
        pltpu.SemaphoreType.DMA,
        pltpu.SemaphoreType.DMA,
        pltpu.SemaphoreType.DMA,
        pltpu.SemaphoreType.DMA,
    ],
    compiler_params=_SC_PARAMS,
)


def _row_spec(w):
    return pl.BlockSpec((BN, w), lambda i: (i, 0))


def _full_spec(shape):
    return pl.BlockSpec(shape, lambda i: tuple(0 for _ in shape))


def _relu(v):
    return jnp.maximum(v, 0.0)


def _dot(a, b):
    return jnp.dot(a, b, preferred_element_type=jnp.float32)


def _emit_tables(hf, hb, outs):
    f0, f1, b0, b1 = outs
    f0[...] = hf[:, :HH]
    f1[...] = hf[:, HH:]
    b0[...] = hb[:, :HH].astype(jnp.bfloat16)
    b1[...] = hb[:, HH:].astype(jnp.bfloat16)


def _enc_body(x, pin, d0, W_se, b_se, W_pe1, b_pe1, W_pe2, b_pe2,
              W_c1, W_c1q, W_pc1, W_pc1q,
              dinv_o, tsf0, tsf1, tsb0, tsb1, tpf0, tpf1, tpb0, tpb1):
    dinv = lax.rsqrt(d0[:, :1])
    s0 = _relu(x[:, 0:1] * W_se[0:1, :] + x[:, 1:2] * W_se[1:2, :] + b_se[...])
    p = _relu(pin[:, 0:1] * W_pe1[0:1, :] + pin[:, 1:2] * W_pe1[1:2, :]
              + b_pe1[...])
    p0 = _relu(_dot(p, W_pe2[...]) + b_pe2[...])
    dinv_o[...] = dinv
    _emit_tables(dinv * _dot(s0, W_c1q[...]), dinv * _dot(s0, W_c1[...]),
                 (tsf0, tsf1, tsb0, tsb1))
    _emit_tables(dinv * _dot(p0, W_pc1q[...]), dinv * _dot(p0, W_pc1[...]),
                 (tpf0, tpf1, tpb0, tpb1))


def _tc_enc(*args):
    return pl.pallas_call(
        _enc_body,
        grid=(GRID,),
        in_specs=[_row_spec(2), _row_spec(2), _row_spec(HH),
                  _full_spec((2, H)), _full_spec((1, H)),
                  _full_spec((2, H)), _full_spec((1, H)),
                  _full_spec((H, H)), _full_spec((1, H)),
                  _full_spec((H, H)), _full_spec((H, H)),
                  _full_spec((H, H)), _full_spec((H, H))],
        out_specs=[_row_spec(1)] + [_row_spec(HH)] * 8,
        out_shape=[jax.ShapeDtypeStruct((NP, 1), jnp.float32)]
        + [jax.ShapeDtypeStruct((NP, HH), jnp.float32)] * 2
        + [jax.ShapeDtypeStruct((NP, HH), jnp.bfloat16)] * 2
        + [jax.ShapeDtypeStruct((NP, HH), jnp.float32)] * 2
        + [jax.ShapeDtypeStruct((NP, HH), jnp.bfloat16)] * 2,
    )(*args)


def _round_body(Ss0, Ss1, Sp0, Sp1, dinv, b_s, b_p,
                W_sg, W_sq, W_pg, W_pq,
                tsf0, tsf1, tsb0, tsb1, tpf0, tpf1, tpb0, tpb1):
    dv = dinv[...]
    s = _relu(dv * jnp.concatenate([Ss0[...], Ss1[...]], axis=1) + b_s[...])
    p = _relu(dv * jnp.concatenate([Sp0[...], Sp1[...]], axis=1) + b_p[...])
    _emit_tables(dv * _dot(s, W_sq[...]), dv * _dot(s, W_sg[...]),
                 (tsf0, tsf1, tsb0, tsb1))
    _emit_tables(dv * _dot(p, W_pq[...]), dv * _dot(p, W_pg[...]),
                 (tpf0, tpf1, tpb0, tpb1))


def _tc_round(*args):
    return pl.pallas_call(
        _round_body,
        grid=(GRID,),
        in_specs=[_row_spec(HH)] * 4 + [_row_spec(1),
                  _full_spec((1, H)), _full_spec((1, H))]
        + [_full_spec((H, H))] * 4,
        out_specs=[_row_spec(HH)] * 8,
        out_shape=([jax.ShapeDtypeStruct((NP, HH), jnp.float32)] * 2
                   + [jax.ShapeDtypeStruct((NP, HH), jnp.bfloat16)] * 2) * 2,
    )(*args)


def _mix_body(Ss0, Ss1, Sp0, Sp1, dinv, b_s, b_p,
              W_tg, W_tq, W_bg, W_bq, tcf0, tcf1, tcb0, tcb1):
    dv = dinv[...]
    s = _relu(dv * jnp.concatenate([Ss0[...], Ss1[...]], axis=1) + b_s[...])
    p = _relu(dv * jnp.concatenate([Sp0[...], Sp1[...]], axis=1) + b_p[...])
    hf = dv * (_dot(s, W_tq[...]) + _dot(p, W_bq[...]))
    hb = dv * (_dot(s, W_tg[...]) + _dot(p, W_bg[...]))
    _emit_tables(hf, hb, (tcf0, tcf1, tcb0, tcb1))


def _tc_mix(*args):
    return pl.pallas_call(
        _mix_body,
        grid=(GRID,),
        in_specs=[_row_spec(HH)] * 4 + [_row_spec(1),
                  _full_spec((1, H)), _full_spec((1, H))]
        + [_full_spec((H, H))] * 4,
        out_specs=[_row_spec(HH)] * 4,
        out_shape=[jax.ShapeDtypeStruct((NP, HH), jnp.float32)] * 2
        + [jax.ShapeDtypeStruct((NP, HH), jnp.bfloat16)] * 2,
    )(*args)


def _fin_body(Sc0, Sc1, dinv, b_cc, W_d1, b_d1, W_d2, b_d2, W_d3, b_d3, out):
    dv = dinv[...]
    c = _relu(dv * jnp.concatenate([Sc0[...], Sc1[...]], axis=1) + b_cc[...])
    d = _relu(_dot(c, W_d1[...]) + b_d1[...])
    d = _relu(_dot(d, W_d2[...]) + b_d2[...])
    d3 = jnp.sum(d * W_d3[...], axis=1, keepdims=True) + b_d3[...]
    out[...] = 1.0 / (1.0 + jnp.exp(-d3))


def _tc_fin(*args):
    return pl.pallas_call(
        _fin_body,
        grid=(GRID,),
        in_specs=[_row_spec(HH)] * 2 + [_row_spec(1),
                  _full_spec((1, H)),
                  _full_spec((H, H)), _full_spec((1, H)),
                  _full_spec((H, HH)), _full_spec((1, HH)),
                  _full_spec((1, HH)), _full_spec((1, 1))],
        out_specs=[_row_spec(1)],
        out_shape=[jax.ShapeDtypeStruct((NP, 1), jnp.float32)],
    )(*args)[0]


@jax.jit
def _run(x, true_alpha_t, true_torque_t, edge_index,
         W_se, b_se, W_pe1, b_pe1, W_pe2, b_pe2,
         W_c1, b_c1, W_c2, b_c2, W_pc1, b_pc1, W_pc2, b_pc2,
         W_cc, b_cc, W_d1, b_d1, W_d2, b_d2, W_d3, b_d3):
    f32 = jnp.float32
    i32 = jnp.int32
    qf = jnp.asarray(_QF)
    xp = jnp.zeros((NP, 2), f32).at[:N].set(x)
    pin = jnp.zeros((NP, 2), f32).at[:N, 0].set(true_alpha_t[:, 0])
    pin = pin.at[:N, 1].set(true_torque_t[:, 0])
    pad = jnp.full((EP - E,), N, i32)
    srcr = jnp.concatenate([edge_index[0], pad]).reshape(NCHUNKS, C, 128)
    dstr = jnp.concatenate([edge_index[1], pad]).reshape(NCHUNKS, C, 128)
    idxc = jnp.stack([srcr, dstr], axis=1)
    ones_b = jnp.ones((NP, HH), jnp.bfloat16)
    ones_f = jnp.ones((NP, HH), f32)

    def row(b):
        return b.reshape(1, -1)

    f_deg = jnp.zeros((16,), i32).at[1].set(1)
    f_dual = jnp.zeros((16,), i32).at[0].set(1)
    f_one = jnp.zeros((16,), i32)

    W_c1q, W_pc1q = W_c1[:, qf], W_pc1[:, qf]
    W_c2g, W_c2q = W_c2[qf, :], W_c2[qf][:, qf]
    W_pc2g, W_pc2q = W_pc2[qf, :], W_pc2[qf][:, qf]
    Wcc_t, Wcc_b = W_cc[:H], W_cc[H:]
    Wcc_tg, Wcc_tq = Wcc_t[qf, :], Wcc_t[qf][:, qf]
    Wcc_bg, Wcc_bq = Wcc_b[qf, :], Wcc_b[qf][:, qf]
    b_c1q, b_pc1q = b_c1[qf], b_pc1[qf]
    b_c2q, b_pc2q = b_c2[qf], b_pc2[qf]
    b_ccq = b_cc[qf]
    W_d1g = W_d1[qf, :]

    d0, _, _, _ = _sc_spmm(ones_b, ones_b, ones_f, ones_f,
                           ones_b, ones_b, ones_f, ones_f, idxc, f_deg)

    (dinv, tsf0, tsf1, tsb0, tsb1, tpf0, tpf1, tpb0, tpb1) = _tc_enc(
        xp, pin, d0, W_se, row(b_se), W_pe1, row(b_pe1), W_pe2, row(b_pe2),
        W_c1, W_c1q, W_pc1, W_pc1q)

    Ss0, Ss1, Sp0, Sp1 = _sc_spmm(tsb0, tsb1, tsf0, tsf1,
                                  tpb0, tpb1, tpf0, tpf1, idxc, f_dual)
    (tsf0, tsf1, tsb0, tsb1, tpf0, tpf1, tpb0, tpb1) = _tc_round(
        Ss0, Ss1, Sp0, Sp1, dinv, row(b_c1q), row(b_pc1q),
        W_c2g, W_c2q, W_pc2g, W_pc2q)

    Ss0, Ss1, Sp0, Sp1 = _sc_spmm(tsb0, tsb1, tsf0, tsf1,
                                  tpb0, tpb1, tpf0, tpf1, idxc, f_dual)
    tcf0, tcf1, tcb0, tcb1 = _tc_mix(
        Ss0, Ss1, Sp0, Sp1, dinv, row(b_c2q), row(b_pc2q),
        Wcc_tg, Wcc_tq, Wcc_bg, Wcc_bq)

    Sc0, Sc1, _, _ = _sc_spmm(tcb0, tcb1, tcf0, tcf1,
                              tcb0, tcb1, tcf0, tcf1, idxc, f_one)
    out = _tc_fin(Sc0, Sc1, dinv, row(b_ccq), W_d1g, row(b_d1),
                  W_d2, row(b_d2), W_d3.reshape(1, HH), b_d3.reshape(1, 1))
    return out[:N]


def kernel(x, true_alpha_t, true_torque_t, edge_index,
           W_se, b_se, W_pe1, b_pe1, W_pe2, b_pe2,
           W_c1, b_c1, W_c2, b_c2, W_pc1, b_pc1, W_pc2, b_pc2,
           W_cc, b_cc, W_d1, b_d1, W_d2, b_d2, W_d3, b_d3):
    return _run(x, true_alpha_t, true_torque_t, edge_index,
                W_se, b_se, W_pe1, b_pe1, W_pe2, b_pe2,
                W_c1, b_c1, W_c2, b_c2, W_pc1, b_pc1, W_pc2, b_pc2,
                W_cc, b_cc, W_d1, b_d1, W_d2, b_d2, W_d3, b_d3)
